# baseline (speedup 1.0000x reference)
import functools

import jax
import jax.numpy as jnp
from jax.experimental import pallas as pl
from jax.experimental.pallas import tpu as pltpu

EMB_DIM = 64
NUM_EMB = 1024
COMMIT = 0.25
ROWS = 16384
TILE = 1024
GRID = ROWS // TILE


def _vq_kernel(xc_ref, e_ref, enc_ref, q_ref, counts_ref, losspart_ref):
    xc = xc_ref[0]
    x = xc.T
    e = e_ref[...]

    x2 = jnp.sum(x * x, axis=1, keepdims=True)
    e2 = jnp.sum(e * e, axis=1)
    xe2 = jax.lax.dot_general(
        x, 2.0 * e, (((1,), (1,)), ((), ())),
        preferred_element_type=jnp.float32)
    d = (x2 + e2[None, :]) - xe2

    dmin = jnp.min(d, axis=1, keepdims=True)
    col = jax.lax.broadcasted_iota(jnp.int32, d.shape, 1).astype(jnp.float32)
    idx = jnp.min(jnp.where(d == dmin, col, float(NUM_EMB)),
                  axis=1, keepdims=True)

    onehot = (col == idx).astype(jnp.float32)
    enc_ref[...] = onehot

    qc = jax.lax.dot_general(
        e, onehot, (((0,), (1,)), ((), ())),
        preferred_element_type=jnp.float32)
    q_ref[0] = xc + (qc - xc)

    diff = qc - xc
    losspart_ref[0, 0, 0] = jnp.sum(diff * diff)
    counts_ref[...] = jnp.sum(onehot, axis=0, keepdims=True)[None]


def _finalize_kernel(counts_ref, losspart_ref, loss_ref, perp_ref):
    acc = losspart_ref[0, 0, 0]
    for k in range(1, GRID):
        acc += losspart_ref[k, 0, 0]
    loss_ref[0, 0] = COMMIT * acc / (ROWS * EMB_DIM)
    p = jnp.sum(counts_ref[...], axis=0) / ROWS
    perp_ref[0, 0] = jnp.exp(-jnp.sum(p * jnp.log(p + 1e-10)))


@functools.partial(jax.jit)
def kernel(inputs, embedding_weight):
    enc, qflat, counts, losspart = pl.pallas_call(
        _vq_kernel,
        grid=(GRID,),
        in_specs=[
            pl.BlockSpec((1, EMB_DIM, TILE), lambda i: (i, 0, 0)),
            pl.BlockSpec((NUM_EMB, EMB_DIM), lambda i: (0, 0)),
        ],
        out_specs=[
            pl.BlockSpec((TILE, NUM_EMB), lambda i: (i, 0)),
            pl.BlockSpec((1, EMB_DIM, TILE), lambda i: (i, 0, 0)),
            pl.BlockSpec((1, 1, NUM_EMB), lambda i: (i, 0, 0)),
            pl.BlockSpec((1, 1, 1), lambda i: (i, 0, 0), memory_space=pltpu.SMEM),
        ],
        out_shape=[
            jax.ShapeDtypeStruct((ROWS, NUM_EMB), jnp.float32),
            jax.ShapeDtypeStruct((16, EMB_DIM, TILE), jnp.float32),
            jax.ShapeDtypeStruct((GRID, 1, NUM_EMB), jnp.float32),
            jax.ShapeDtypeStruct((GRID, 1, 1), jnp.float32),
        ],
        compiler_params=pltpu.CompilerParams(
            dimension_semantics=("parallel",)),
    )(inputs.reshape(16, EMB_DIM, TILE), embedding_weight)

    loss, perp = pl.pallas_call(
        _finalize_kernel,
        in_specs=[
            pl.BlockSpec((GRID, 1, NUM_EMB), lambda: (0, 0, 0)),
            pl.BlockSpec((GRID, 1, 1), lambda: (0, 0, 0), memory_space=pltpu.SMEM),
        ],
        out_specs=[
            pl.BlockSpec((1, 1), lambda: (0, 0), memory_space=pltpu.SMEM),
            pl.BlockSpec((1, 1), lambda: (0, 0), memory_space=pltpu.SMEM),
        ],
        out_shape=[
            jax.ShapeDtypeStruct((1, 1), jnp.float32),
            jax.ShapeDtypeStruct((1, 1), jnp.float32),
        ],
    )(counts, losspart)

    quantized_out = qflat.reshape(inputs.shape)
    return (loss[0, 0], quantized_out, perp[0, 0], enc)


# speedup vs baseline: 1.0553x; 1.0553x over previous
import functools

import jax
import jax.numpy as jnp
from jax.experimental import pallas as pl
from jax.experimental.pallas import tpu as pltpu

EMB_DIM = 64
NUM_EMB = 1024
COMMIT = 0.25
ROWS = 16384
TILE = 2048
GRID = ROWS // TILE


def _vq_kernel(xc_ref, e_ref, enc_ref, q_ref, counts_ref, losspart_ref):
    xc = jnp.concatenate([xc_ref[0], xc_ref[1]], axis=1)
    x = xc.T
    e = e_ref[...]

    x2 = jnp.sum(x * x, axis=1, keepdims=True)
    e2 = jnp.sum(e * e, axis=1)
    xe2 = jax.lax.dot_general(
        x, 2.0 * e, (((1,), (1,)), ((), ())),
        preferred_element_type=jnp.float32)
    d = (x2 + e2[None, :]) - xe2

    dmin = jnp.min(d, axis=1, keepdims=True)
    col = jax.lax.broadcasted_iota(jnp.int32, d.shape, 1).astype(jnp.float32)
    idx = jnp.min(jnp.where(d == dmin, col, float(NUM_EMB)),
                  axis=1, keepdims=True)

    onehot = (col == idx).astype(jnp.float32)
    enc_ref[...] = onehot

    qc = jax.lax.dot_general(
        e, onehot, (((0,), (1,)), ((), ())),
        preferred_element_type=jnp.float32)
    qst = xc + (qc - xc)
    q_ref[0] = qst[:, :1024]
    q_ref[1] = qst[:, 1024:]

    diff = qc - xc
    losspart_ref[0, 0, 0] = jnp.sum(diff * diff)
    counts_ref[...] = jnp.sum(onehot, axis=0, keepdims=True)[None]


def _finalize_kernel(counts_ref, losspart_ref, loss_ref, perp_ref):
    acc = losspart_ref[0, 0, 0]
    for k in range(1, GRID):
        acc += losspart_ref[k, 0, 0]
    loss_ref[0, 0] = COMMIT * acc / (ROWS * EMB_DIM)
    p = jnp.sum(counts_ref[...], axis=0) / ROWS
    perp_ref[0, 0] = jnp.exp(-jnp.sum(p * jnp.log(p + 1e-10)))


@functools.partial(jax.jit)
def kernel(inputs, embedding_weight):
    enc, qflat, counts, losspart = pl.pallas_call(
        _vq_kernel,
        grid=(GRID,),
        in_specs=[
            pl.BlockSpec((2, EMB_DIM, 1024), lambda i: (i, 0, 0)),
            pl.BlockSpec((NUM_EMB, EMB_DIM), lambda i: (0, 0)),
        ],
        out_specs=[
            pl.BlockSpec((TILE, NUM_EMB), lambda i: (i, 0)),
            pl.BlockSpec((2, EMB_DIM, 1024), lambda i: (i, 0, 0)),
            pl.BlockSpec((1, 1, NUM_EMB), lambda i: (i, 0, 0)),
            pl.BlockSpec((1, 1, 1), lambda i: (i, 0, 0), memory_space=pltpu.SMEM),
        ],
        out_shape=[
            jax.ShapeDtypeStruct((ROWS, NUM_EMB), jnp.float32),
            jax.ShapeDtypeStruct((16, EMB_DIM, 1024), jnp.float32),
            jax.ShapeDtypeStruct((GRID, 1, NUM_EMB), jnp.float32),
            jax.ShapeDtypeStruct((GRID, 1, 1), jnp.float32),
        ],
        compiler_params=pltpu.CompilerParams(
            dimension_semantics=("parallel",)),
    )(inputs.reshape(16, EMB_DIM, 1024), embedding_weight)

    loss, perp = pl.pallas_call(
        _finalize_kernel,
        in_specs=[
            pl.BlockSpec((GRID, 1, NUM_EMB), lambda: (0, 0, 0)),
            pl.BlockSpec((GRID, 1, 1), lambda: (0, 0, 0), memory_space=pltpu.SMEM),
        ],
        out_specs=[
            pl.BlockSpec((1, 1), lambda: (0, 0), memory_space=pltpu.SMEM),
            pl.BlockSpec((1, 1), lambda: (0, 0), memory_space=pltpu.SMEM),
        ],
        out_shape=[
            jax.ShapeDtypeStruct((1, 1), jnp.float32),
            jax.ShapeDtypeStruct((1, 1), jnp.float32),
        ],
    )(counts, losspart)

    quantized_out = qflat.reshape(inputs.shape)
    return (loss[0, 0], quantized_out, perp[0, 0], enc)
